# baseline (speedup 1.0000x reference)
import jax
import jax.numpy as jnp
from jax.experimental import pallas as pl
from jax.experimental.pallas import tpu as pltpu

_VMEM_LIMIT = 48 * 1024 * 1024


def kernel(x, weight, bias):
    n, c, h, w = x.shape
    cout = weight.shape[0]
    ho, wo = h // 2, w // 2
    s = ho * wo
    bf16 = jnp.bfloat16
    m = 2 if n % 2 == 0 else 1

    xf = jnp.transpose(x, (0, 2, 3, 1))
    wt = jnp.transpose(weight, (2, 3, 1, 0))
    wm = wt.reshape(9 * c, cout).astype(bf16)
    ep = 0.25 * jnp.eye(c, dtype=bf16)
    b2 = bias.reshape(cout, 1).astype(jnp.float32)

    def body(x_ref, w_ref, ep_ref, b_ref, yc_ref, yp_ref):
        x6 = (x_ref[...].astype(bf16)
              .reshape(m, h, wo, 2 * c)
              .reshape(m, ho, 2, wo, 2 * c))
        zrow = jnp.zeros((1, wo, c), bf16)
        zcol = jnp.zeros((ho, 1, c), bf16)

        def tap_for(img, ky, kx):
            dy, py = ((-1, 1) if ky == 0 else (0, ky - 1))
            dx, px = ((-1, 1) if kx == 0 else (0, kx - 1))
            a = x6[img, :, py, :, px * c:(px + 1) * c]
            if dy:
                a = jnp.concatenate([zrow, a[0:ho - 1]], axis=0)
            if dx:
                a = jnp.concatenate([zcol, a[:, 0:wo - 1, :]], axis=1)
            return a.reshape(s, c)

        for img in range(m):
            acc = None
            pacc = None
            for ky in range(3):
                for kx in range(3):
                    tap = tap_for(img, ky, kx)
                    i = ky * 3 + kx
                    d = jax.lax.dot_general(w_ref[i * c:(i + 1) * c], tap,
                                            (((0,), (1,)), ((), ())),
                                            preferred_element_type=jnp.float32)
                    acc = d if acc is None else acc + d
                    if ky >= 1 and kx >= 1:
                        p = jax.lax.dot_general(ep_ref[...], tap,
                                                (((0,), (1,)), ((), ())),
                                                preferred_element_type=jnp.float32)
                        pacc = p if pacc is None else pacc + p
            yc_ref[img] = acc + b_ref[...]
            yp_ref[img] = pacc

    yc, yp = pl.pallas_call(
        body,
        out_shape=(jax.ShapeDtypeStruct((n, cout, s), jnp.float32),
                   jax.ShapeDtypeStruct((n, c, s), jnp.float32)),
        grid=(n // m,),
        in_specs=[
            pl.BlockSpec((m, h, w, c), lambda i: (i, 0, 0, 0)),
            pl.BlockSpec((9 * c, cout), lambda i: (0, 0)),
            pl.BlockSpec((c, c), lambda i: (0, 0)),
            pl.BlockSpec((cout, 1), lambda i: (0, 0)),
        ],
        out_specs=(pl.BlockSpec((m, cout, s), lambda i: (i, 0, 0)),
                   pl.BlockSpec((m, c, s), lambda i: (i, 0, 0))),
        compiler_params=pltpu.CompilerParams(
            dimension_semantics=("parallel",),
            vmem_limit_bytes=_VMEM_LIMIT,
        ),
        cost_estimate=pl.CostEstimate(
            flops=2 * n * s * (9 + 4) * c * cout,
            transcendentals=0,
            bytes_accessed=(n * c * h * w * 2 + 9 * c * cout * 2
                            + n * s * (c + cout) * 4),
        ),
    )(xf, wm, ep, b2)

    return yc.reshape(n, cout, ho, wo), yp.reshape(n, c, ho, wo)


# speedup vs baseline: 1.0158x; 1.0158x over previous
import jax
import jax.numpy as jnp
from jax.experimental import pallas as pl
from jax.experimental.pallas import tpu as pltpu

_VMEM_LIMIT = 48 * 1024 * 1024


def kernel(x, weight, bias):
    n, c, h, w = x.shape
    cout = weight.shape[0]
    ho, wo = h // 2, w // 2
    s = ho * wo
    bf16 = jnp.bfloat16
    m = 4 if n % 4 == 0 else (2 if n % 2 == 0 else 1)

    xf = jnp.transpose(x, (0, 2, 3, 1))
    wt = jnp.transpose(weight, (2, 3, 1, 0))
    wm = wt.reshape(9 * c, cout).astype(bf16)
    ep = 0.25 * jnp.eye(c, dtype=bf16)
    b2 = bias.reshape(cout, 1).astype(jnp.float32)

    def body(xa_ref, xb_ref, w_ref, ep_ref, b_ref, yc_ref, yp_ref):
        def fold(r):
            return (r[...].astype(bf16)
                    .reshape(m // 2, h, wo, 2 * c)
                    .reshape(m // 2, ho, 2, wo, 2 * c))
        x6 = jnp.concatenate([fold(xa_ref), fold(xb_ref)], axis=0)
        zrow = jnp.zeros((1, wo, c), bf16)
        zcol = jnp.zeros((ho, 1, c), bf16)

        def tap_for(img, ky, kx):
            dy, py = ((-1, 1) if ky == 0 else (0, ky - 1))
            dx, px = ((-1, 1) if kx == 0 else (0, kx - 1))
            a = x6[img, :, py, :, px * c:(px + 1) * c]
            if dy:
                a = jnp.concatenate([zrow, a[0:ho - 1]], axis=0)
            if dx:
                a = jnp.concatenate([zcol, a[:, 0:wo - 1, :]], axis=1)
            return a.reshape(s, c)

        for img in range(m):
            acc = None
            pacc = None
            for ky in range(3):
                for kx in range(3):
                    tap = tap_for(img, ky, kx)
                    i = ky * 3 + kx
                    d = jax.lax.dot_general(w_ref[i * c:(i + 1) * c], tap,
                                            (((0,), (1,)), ((), ())),
                                            preferred_element_type=jnp.float32)
                    acc = d if acc is None else acc + d
                    if ky >= 1 and kx >= 1:
                        p = jax.lax.dot_general(ep_ref[...], tap,
                                                (((0,), (1,)), ((), ())),
                                                preferred_element_type=jnp.float32)
                        pacc = p if pacc is None else pacc + p
            yc_ref[img] = acc + b_ref[...]
            yp_ref[img] = pacc

    yc, yp = pl.pallas_call(
        body,
        out_shape=(jax.ShapeDtypeStruct((n, cout, s), jnp.float32),
                   jax.ShapeDtypeStruct((n, c, s), jnp.float32)),
        grid=(n // m,),
        in_specs=[
            pl.BlockSpec((m // 2, h, w, c), lambda i: (2 * i, 0, 0, 0)),
            pl.BlockSpec((m // 2, h, w, c), lambda i: (2 * i + 1, 0, 0, 0)),
            pl.BlockSpec((9 * c, cout), lambda i: (0, 0)),
            pl.BlockSpec((c, c), lambda i: (0, 0)),
            pl.BlockSpec((cout, 1), lambda i: (0, 0)),
        ],
        out_specs=(pl.BlockSpec((m, cout, s), lambda i: (i, 0, 0)),
                   pl.BlockSpec((m, c, s), lambda i: (i, 0, 0))),
        compiler_params=pltpu.CompilerParams(
            dimension_semantics=("parallel",),
            vmem_limit_bytes=_VMEM_LIMIT,
        ),
        cost_estimate=pl.CostEstimate(
            flops=2 * n * s * (9 + 4) * c * cout,
            transcendentals=0,
            bytes_accessed=(n * c * h * w * 2 + 9 * c * cout * 2
                            + n * s * (c + cout) * 4),
        ),
    )(xf, xf, wm, ep, b2)

    return yc.reshape(n, cout, ho, wo), yp.reshape(n, c, ho, wo)
